# Initial kernel scaffold; baseline (speedup 1.0000x reference)
import jax
import jax.numpy as jnp
from jax.experimental import pallas as pl


def kernel(x, Wg, bg, W1, b1, W2, b2):
    raise NotImplementedError("write your pallas kernel here")


# speedup vs baseline: 2.2014x; 2.2014x over previous
import functools

import jax
import jax.numpy as jnp
from jax import lax
from jax.experimental import pallas as pl
from jax.experimental.pallas import tpu as pltpu
from jax.experimental.pallas import tpu_sc as plsc

D = 768
H = 768
E = 64
K = 2
T = 2048
BM = 128
NB = 96
PAD = NB * BM
NC, NS = 2, 16
NW = NC * NS
CH = 128
TPW = T // NW


def _router_body(logits_ref,
                 pos0_ref, pos1_ref, w0_ref, w1_ref, gid_ref, nbu_ref,
                 nbu16_ref, lbal_ref, oh0_scr, oh1_scr):
    logits = logits_ref[...]

    mx = jnp.max(logits, axis=1, keepdims=True)
    ex = jnp.exp(logits - mx)
    probs = ex / jnp.sum(ex, axis=1, keepdims=True)
    p_mean = jnp.mean(probs, axis=0, keepdims=True)

    lane = lax.broadcasted_iota(jnp.int32, (T, E), 1)
    is0 = logits == mx
    e0 = jnp.min(jnp.where(is0, lane, E), axis=1, keepdims=True)
    oh0 = lane == e0
    masked = jnp.where(oh0, -jnp.inf, logits)
    m1 = jnp.max(masked, axis=1, keepdims=True)
    is1 = masked == m1
    e1 = jnp.min(jnp.where(is1, lane, E), axis=1, keepdims=True)
    oh1 = lane == e1

    w0 = 1.0 / (1.0 + jnp.exp(m1 - mx))
    w0_ref[...] = w0
    w1_ref[...] = 1.0 - w0

    oh0f = oh0.astype(jnp.float32)
    oh1f = oh1.astype(jnp.float32)
    oh0_scr[...] = oh0f
    oh1_scr[...] = oh1f
    cnt0 = jnp.sum(oh0f, axis=0, keepdims=True)
    cnt = cnt0 + jnp.sum(oh1f, axis=0, keepdims=True)

    f = cnt / float(T * K)
    lbal_ref[...] = float(E) * jnp.sum(f * p_mean, keepdims=True)

    nb = jnp.floor((cnt + float(BM - 1)) * (1.0 / BM))
    tri_e = (lax.broadcasted_iota(jnp.int32, (E, E), 0)
             < lax.broadcasted_iota(jnp.int32, (E, E), 1)).astype(jnp.float32)
    bb = jnp.dot(nb, tri_e, preferred_element_type=jnp.float32,
                 precision=lax.Precision.HIGHEST)
    nbu = jnp.sum(nb, keepdims=True)
    nbu_ref[...] = nbu.astype(jnp.int32)
    lane16 = lax.broadcasted_iota(jnp.int32, (1, 16), 1)
    nbu16_ref[...] = jnp.where(lane16 == 0, nbu.astype(jnp.int32), 0)
    poff = bb * float(BM)

    ends = (bb + nb).astype(jnp.int32)
    b_iota = lax.broadcasted_iota(jnp.int32, (NB, E), 0)
    cap = jnp.minimum(b_iota, nbu.astype(jnp.int32) - 1)
    gid = jnp.sum((jnp.broadcast_to(ends, (NB, E)) <= cap).astype(jnp.int32),
                  axis=1, keepdims=True)
    gid_ref[...] = gid

    tri_m = (lax.broadcasted_iota(jnp.int32, (BM, BM), 1)
             < lax.broadcasted_iota(jnp.int32, (BM, BM), 0)).astype(jnp.float32)
    add0 = poff
    add1 = poff + cnt0

    def chunk(i, carry):
        b0, b1 = carry
        sl = pl.ds(i * BM, BM)
        m0c = oh0_scr[sl, :]
        m1c = oh1_scr[sl, :]
        r0 = jnp.dot(tri_m, m0c, preferred_element_type=jnp.float32,
                     precision=lax.Precision.HIGHEST) + b0
        r1 = jnp.dot(tri_m, m1c, preferred_element_type=jnp.float32,
                     precision=lax.Precision.HIGHEST) + b1
        p0 = jnp.sum((r0 + add0) * m0c, axis=1, keepdims=True)
        p1 = jnp.sum((r1 + add1) * m1c, axis=1, keepdims=True)
        pos0_ref[sl, :] = p0.astype(jnp.int32)
        pos1_ref[sl, :] = p1.astype(jnp.int32)
        return (b0 + jnp.sum(m0c, axis=0, keepdims=True),
                b1 + jnp.sum(m1c, axis=0, keepdims=True))

    zero = jnp.zeros((1, E), jnp.float32)
    lax.fori_loop(0, T // BM, chunk, (zero, zero))


def _router(logits):
    out_shapes = [
        jax.ShapeDtypeStruct((T, 1), jnp.int32),
        jax.ShapeDtypeStruct((T, 1), jnp.int32),
        jax.ShapeDtypeStruct((T, 1), jnp.float32),
        jax.ShapeDtypeStruct((T, 1), jnp.float32),
        jax.ShapeDtypeStruct((NB, 1), jnp.int32),
        jax.ShapeDtypeStruct((1, 1), jnp.int32),
        jax.ShapeDtypeStruct((1, 16), jnp.int32),
        jax.ShapeDtypeStruct((1, 1), jnp.float32),
    ]
    return pl.pallas_call(
        _router_body,
        out_shape=out_shapes,
        scratch_shapes=[pltpu.VMEM((T, E), jnp.float32),
                        pltpu.VMEM((T, E), jnp.float32)],
    )(logits)


def _ffn_body(gid_ref, nbu_ref, x_ref, w1_ref, b1_ref, w2_ref, b2_ref, pw_ref,
              y_ref):
    b = pl.program_id(0)

    @pl.when(b < nbu_ref[0])
    def _():
        xb = x_ref[...]
        h = jnp.maximum(
            jnp.dot(xb, w1_ref[0], preferred_element_type=jnp.float32)
            + b1_ref[0], 0.0)
        y = (jnp.dot(h, w2_ref[0], preferred_element_type=jnp.float32)
             + b2_ref[0])
        y_ref[...] = y * pw_ref[...]


def _ffn(gid, nbu, xpad, W1, b1, W2, b2, pwgt):
    grid_spec = pltpu.PrefetchScalarGridSpec(
        num_scalar_prefetch=2,
        grid=(NB,),
        in_specs=[
            pl.BlockSpec((BM, D), lambda b, gid, nbu: (b, 0)),
            pl.BlockSpec((1, D, H), lambda b, gid, nbu: (gid[b], 0, 0)),
            pl.BlockSpec((1, 1, H), lambda b, gid, nbu: (gid[b], 0, 0)),
            pl.BlockSpec((1, H, D), lambda b, gid, nbu: (gid[b], 0, 0)),
            pl.BlockSpec((1, 1, D), lambda b, gid, nbu: (gid[b], 0, 0)),
            pl.BlockSpec((BM, 1), lambda b, gid, nbu: (b, 0)),
        ],
        out_specs=pl.BlockSpec((BM, D), lambda b, gid, nbu: (b, 0)),
    )
    return pl.pallas_call(
        _ffn_body,
        grid_spec=grid_spec,
        out_shape=jax.ShapeDtypeStruct((PAD, D), jnp.float32),
        compiler_params=pltpu.CompilerParams(
            dimension_semantics=("arbitrary",)),
    )(gid, nbu, xpad, W1, b1.reshape(E, 1, H), W2, b2.reshape(E, 1, D), pwgt)


@functools.cache
def _sc_kernels():
    mesh = plsc.VectorSubcoreMesh(core_axis_name="c", subcore_axis_name="s")

    def wid():
        return lax.axis_index("s") * NC + lax.axis_index("c")

    @functools.partial(
        pl.kernel, mesh=mesh,
        out_type=[jax.ShapeDtypeStruct((PAD,), jnp.int32),
                  jax.ShapeDtypeStruct((PAD,), jnp.float32)],
        scratch_types=[pltpu.VMEM((BM,), jnp.int32),
                       pltpu.VMEM((BM,), jnp.int32),
                       pltpu.VMEM((BM,), jnp.float32)])
    def scatter_sc(pos0_hbm, pos1_hbm, w0_hbm, w1_hbm, tok_hbm,
                   ptok_hbm, pwgt_hbm, idx_v, tok_v, val_v):
        w = wid()
        c = w // 2
        sl = pl.ds(c * BM, BM)
        pltpu.sync_copy(tok_hbm.at[sl], tok_v)

        @pl.when(w % 2 == 0)
        def _():
            pltpu.sync_copy(pos0_hbm.at[sl], idx_v)
            pltpu.sync_copy(w0_hbm.at[sl], val_v)

        @pl.when(w % 2 == 1)
        def _():
            pltpu.sync_copy(pos1_hbm.at[sl], idx_v)
            pltpu.sync_copy(w1_hbm.at[sl], val_v)

        pltpu.sync_copy(tok_v, ptok_hbm.at[idx_v])
        pltpu.sync_copy(val_v, pwgt_hbm.at[idx_v])

    @functools.partial(
        pl.kernel, mesh=mesh,
        out_type=jax.ShapeDtypeStruct((PAD, D), jnp.float32),
        scratch_types=[pltpu.VMEM((CH,), jnp.int32),
                       pltpu.VMEM((CH,), jnp.int32),
                       pltpu.VMEM((CH, D), jnp.float32),
                       pltpu.SemaphoreType.DMA])
    def gather_sc(ptok_hbm, x_hbm, xpad_hbm, idx_v, idx2_v, rows_v, sem):
        w = wid()

        def chunk(j, carry):
            base = (w + NW * j) * CH
            pltpu.sync_copy(ptok_hbm.at[pl.ds(base, CH)], idx_v)

            def clamp(i, carry2):
                s16 = pl.ds(i * 16, 16)
                idx2_v[s16] = jnp.clip(idx_v[s16], 0, T - 1)
                return carry2

            lax.fori_loop(0, CH // 16, clamp, 0)
            pltpu.async_copy(x_hbm.at[idx2_v], rows_v, sem).wait()
            pltpu.sync_copy(rows_v, xpad_hbm.at[pl.ds(base, CH)])
            return carry

        lax.fori_loop(0, PAD // (NW * CH), chunk, 0)

    @functools.partial(
        pl.kernel, mesh=mesh,
        out_type=jax.ShapeDtypeStruct((T, D), jnp.float32),
        scratch_types=[pltpu.VMEM((TPW,), jnp.int32),
                       pltpu.VMEM((TPW,), jnp.int32),
                       pltpu.VMEM((TPW, D), jnp.float32),
                       pltpu.VMEM((TPW, D), jnp.float32),
                       pltpu.SemaphoreType.DMA,
                       pltpu.SemaphoreType.DMA])
    def combine_sc(y_hbm, pos0_hbm, pos1_hbm, out_hbm, i0_v, i1_v, r0_v, r1_v,
                   sem0, sem1):
        base = wid() * TPW
        pltpu.sync_copy(pos0_hbm.at[pl.ds(base, TPW)], i0_v)
        pltpu.sync_copy(pos1_hbm.at[pl.ds(base, TPW)], i1_v)
        c0 = pltpu.async_copy(y_hbm.at[i0_v], r0_v, sem0)
        c1 = pltpu.async_copy(y_hbm.at[i1_v], r1_v, sem1)
        c0.wait()
        c1.wait()

        def row(r, carry):
            for c in range(D // 16):
                sl = pl.ds(c * 16, 16)
                r0_v[r, sl] = r0_v[r, sl] + r1_v[r, sl]
            return carry

        lax.fori_loop(0, TPW, row, 0)
        pltpu.sync_copy(r0_v, out_hbm.at[pl.ds(base, TPW)])

    return scatter_sc, gather_sc, combine_sc


def kernel(x, Wg, bg, W1, b1, W2, b2):
    xf = x.reshape(T, D)
    logits = xf @ Wg + bg
    pos0, pos1, w0, w1, gid, nbu, nbu16, lbal = _router(logits)
    pos0 = pos0.reshape(-1)
    pos1 = pos1.reshape(-1)

    scatter_sc, gather_sc, combine_sc = _sc_kernels()
    tok = jnp.arange(T, dtype=jnp.int32)
    ptok, pwgt = scatter_sc(pos0, pos1, w0.reshape(-1), w1.reshape(-1), tok)
    xpad = gather_sc(ptok, xf)

    y = _ffn(gid.reshape(-1), nbu.reshape(-1), xpad, W1, b1, W2, b2,
             pwgt.reshape(PAD, 1))

    out = combine_sc(y, pos0, pos1).reshape(x.shape)
    return out, lbal[0, 0]


# speedup vs baseline: 2.5275x; 1.1481x over previous
import functools

import jax
import jax.numpy as jnp
from jax import lax
from jax.experimental import pallas as pl
from jax.experimental.pallas import tpu as pltpu
from jax.experimental.pallas import tpu_sc as plsc

D = 768
H = 768
E = 64
K = 2
T = 2048
BM = 64
NB = 128
PAD = NB * BM
PADH = PAD // 2
NBH = NB // 2
CS = 128
NC, NS = 2, 16
NW = NC * NS
CH = 128
SCH = T // (NW // K)
TPW = T // NW


def _router_body(logits_ref,
                 pos0_ref, pos1_ref, w0_ref, w1_ref, gid_ref, nbu_ref,
                 nbu16_ref, lbal_ref, oh0_scr, oh1_scr):
    logits = logits_ref[...]

    mx = jnp.max(logits, axis=1, keepdims=True)
    ex = jnp.exp(logits - mx)
    probs = ex / jnp.sum(ex, axis=1, keepdims=True)
    p_mean = jnp.mean(probs, axis=0, keepdims=True)

    lane = lax.broadcasted_iota(jnp.int32, (T, E), 1)
    is0 = logits == mx
    e0 = jnp.min(jnp.where(is0, lane, E), axis=1, keepdims=True)
    oh0 = lane == e0
    masked = jnp.where(oh0, -jnp.inf, logits)
    m1 = jnp.max(masked, axis=1, keepdims=True)
    is1 = masked == m1
    e1 = jnp.min(jnp.where(is1, lane, E), axis=1, keepdims=True)
    oh1 = lane == e1

    w0 = 1.0 / (1.0 + jnp.exp(m1 - mx))
    w0_ref[...] = w0
    w1_ref[...] = 1.0 - w0

    oh0f = oh0.astype(jnp.float32)
    oh1f = oh1.astype(jnp.float32)
    oh0_scr[...] = oh0f
    oh1_scr[...] = oh1f
    cnt0 = jnp.sum(oh0f, axis=0, keepdims=True)
    cnt = cnt0 + jnp.sum(oh1f, axis=0, keepdims=True)

    f = cnt / float(T * K)
    lbal_ref[...] = float(E) * jnp.sum(f * p_mean, keepdims=True)

    nb = jnp.floor((cnt + float(BM - 1)) * (1.0 / BM))
    tri_e = (lax.broadcasted_iota(jnp.int32, (E, E), 0)
             < lax.broadcasted_iota(jnp.int32, (E, E), 1)).astype(jnp.float32)
    bb = jnp.dot(nb, tri_e, preferred_element_type=jnp.float32,
                 precision=lax.Precision.HIGHEST)
    nbu = jnp.sum(nb, keepdims=True)
    nbu_ref[...] = nbu.astype(jnp.int32)
    lane16 = lax.broadcasted_iota(jnp.int32, (1, 16), 1)
    nbu16_ref[...] = jnp.where(lane16 == 0, nbu.astype(jnp.int32), 0)
    poff = bb * float(BM)

    ends = (bb + nb).astype(jnp.int32)
    b_iota = lax.broadcasted_iota(jnp.int32, (NB, E), 0)
    cap = jnp.minimum(b_iota, nbu.astype(jnp.int32) - 1)
    gid = jnp.sum((jnp.broadcast_to(ends, (NB, E)) <= cap).astype(jnp.int32),
                  axis=1, keepdims=True)
    gid_ref[...] = gid

    tri_m = (lax.broadcasted_iota(jnp.int32, (CS, CS), 1)
             < lax.broadcasted_iota(jnp.int32, (CS, CS), 0)).astype(jnp.float32)
    add0 = poff
    add1 = poff + cnt0

    def chunk(i, carry):
        b0, b1 = carry
        sl = pl.ds(i * CS, CS)
        m0c = oh0_scr[sl, :]
        m1c = oh1_scr[sl, :]
        r0 = jnp.dot(tri_m, m0c, preferred_element_type=jnp.float32,
                     precision=lax.Precision.HIGHEST) + b0
        r1 = jnp.dot(tri_m, m1c, preferred_element_type=jnp.float32,
                     precision=lax.Precision.HIGHEST) + b1
        p0 = jnp.sum((r0 + add0) * m0c, axis=1, keepdims=True)
        p1 = jnp.sum((r1 + add1) * m1c, axis=1, keepdims=True)
        pos0_ref[sl, :] = p0.astype(jnp.int32)
        pos1_ref[sl, :] = p1.astype(jnp.int32)
        return (b0 + jnp.sum(m0c, axis=0, keepdims=True),
                b1 + jnp.sum(m1c, axis=0, keepdims=True))

    zero = jnp.zeros((1, E), jnp.float32)
    lax.fori_loop(0, T // CS, chunk, (zero, zero))


def _router(logits):
    out_shapes = [
        jax.ShapeDtypeStruct((T, 1), jnp.int32),
        jax.ShapeDtypeStruct((T, 1), jnp.int32),
        jax.ShapeDtypeStruct((T, 1), jnp.float32),
        jax.ShapeDtypeStruct((T, 1), jnp.float32),
        jax.ShapeDtypeStruct((NB, 1), jnp.int32),
        jax.ShapeDtypeStruct((1, 1), jnp.int32),
        jax.ShapeDtypeStruct((1, 16), jnp.int32),
        jax.ShapeDtypeStruct((1, 1), jnp.float32),
    ]
    return pl.pallas_call(
        _router_body,
        out_shape=out_shapes,
        scratch_shapes=[pltpu.VMEM((T, E), jnp.float32),
                        pltpu.VMEM((T, E), jnp.float32)],
    )(logits)


def _ffn_body(off, *refs):
    (gid_ref, nbu_ref, x_ref, w1_ref, b1_ref, w2_ref, b2_ref, pw_ref) = refs[:8]
    y_ref = refs[-1]
    b = pl.program_id(0)

    @pl.when(b + off < nbu_ref[0])
    def _():
        xb = x_ref[...]
        h = jnp.maximum(
            jnp.dot(xb, w1_ref[0], preferred_element_type=jnp.float32)
            + b1_ref[0], 0.0)
        y = (jnp.dot(h, w2_ref[0], preferred_element_type=jnp.float32)
             + b2_ref[0])
        y_ref[...] = y * pw_ref[...]


def _ffn_half(off, gid, nbu, xpad_h, W1, b1, W2, b2, pwgt_h, y_in=None):
    def hclamp(b, nbu):
        return jnp.minimum(b, jnp.maximum(nbu[0] - 1 - off, 0))

    in_specs = [
        pl.BlockSpec((BM, D), lambda b, gid, nbu: (hclamp(b, nbu), 0)),
        pl.BlockSpec((1, D, H), lambda b, gid, nbu: (gid[b], 0, 0)),
        pl.BlockSpec((1, 1, H), lambda b, gid, nbu: (gid[b], 0, 0)),
        pl.BlockSpec((1, H, D), lambda b, gid, nbu: (gid[b], 0, 0)),
        pl.BlockSpec((1, 1, D), lambda b, gid, nbu: (gid[b], 0, 0)),
        pl.BlockSpec((BM, 1), lambda b, gid, nbu: (hclamp(b, nbu), 0)),
    ]
    args = [gid, nbu, xpad_h, W1, b1.reshape(E, 1, H), W2,
            b2.reshape(E, 1, D), pwgt_h]
    aliases = {}
    if y_in is not None:
        in_specs.append(pl.BlockSpec(memory_space=pltpu.HBM))
        args.append(y_in)
        aliases = {8: 0}

    grid_spec = pltpu.PrefetchScalarGridSpec(
        num_scalar_prefetch=2,
        grid=(NBH,),
        in_specs=in_specs,
        out_specs=pl.BlockSpec((BM, D), lambda b, gid, nbu: (off + b, 0)),
    )
    return pl.pallas_call(
        functools.partial(_ffn_body, off),
        grid_spec=grid_spec,
        out_shape=jax.ShapeDtypeStruct((PAD, D), jnp.float32),
        input_output_aliases=aliases,
        compiler_params=pltpu.CompilerParams(
            dimension_semantics=("arbitrary",)),
    )(*args)


@functools.cache
def _sc_kernels():
    mesh = plsc.VectorSubcoreMesh(core_axis_name="c", subcore_axis_name="s")

    def wid():
        return lax.axis_index("s") * NC + lax.axis_index("c")

    @functools.partial(
        pl.kernel, mesh=mesh,
        out_type=[jax.ShapeDtypeStruct((PAD,), jnp.int32),
                  jax.ShapeDtypeStruct((PAD,), jnp.float32)],
        scratch_types=[pltpu.VMEM((SCH,), jnp.int32),
                       pltpu.VMEM((SCH,), jnp.int32),
                       pltpu.VMEM((SCH,), jnp.float32)])
    def scatter_sc(pos0_hbm, pos1_hbm, w0_hbm, w1_hbm, tok_hbm,
                   ptok_hbm, pwgt_hbm, idx_v, tok_v, val_v):
        w = wid()
        c = w // 2
        sl = pl.ds(c * SCH, SCH)
        pltpu.sync_copy(tok_hbm.at[sl], tok_v)

        @pl.when(w % 2 == 0)
        def _():
            pltpu.sync_copy(pos0_hbm.at[sl], idx_v)
            pltpu.sync_copy(w0_hbm.at[sl], val_v)

        @pl.when(w % 2 == 1)
        def _():
            pltpu.sync_copy(pos1_hbm.at[sl], idx_v)
            pltpu.sync_copy(w1_hbm.at[sl], val_v)

        pltpu.sync_copy(tok_v, ptok_hbm.at[idx_v])
        pltpu.sync_copy(val_v, pwgt_hbm.at[idx_v])

    def make_gather(off):
        @functools.partial(
            pl.kernel, mesh=mesh,
            out_type=jax.ShapeDtypeStruct((PADH, D), jnp.float32),
            scratch_types=[pltpu.VMEM((CH,), jnp.int32),
                           pltpu.VMEM((CH,), jnp.int32),
                           pltpu.VMEM((CH, D), jnp.float32),
                           pltpu.SemaphoreType.DMA])
        def gather_half(ptok_hbm, x_hbm, xpad_hbm, idx_v, idx2_v, rows_v,
                        sem):
            base = wid() * CH
            pltpu.sync_copy(ptok_hbm.at[pl.ds(off + base, CH)], idx_v)

            def clamp(i, carry):
                s16 = pl.ds(i * 16, 16)
                idx2_v[s16] = jnp.clip(idx_v[s16], 0, T - 1)
                return carry

            lax.fori_loop(0, CH // 16, clamp, 0)
            pltpu.async_copy(x_hbm.at[idx2_v], rows_v, sem).wait()
            pltpu.sync_copy(rows_v, xpad_hbm.at[pl.ds(base, CH)])

        return gather_half

    gather0_sc = make_gather(0)
    gather1_sc = make_gather(PADH)

    @functools.partial(
        pl.kernel, mesh=mesh,
        out_type=jax.ShapeDtypeStruct((T, D), jnp.float32),
        scratch_types=[pltpu.VMEM((TPW,), jnp.int32),
                       pltpu.VMEM((TPW,), jnp.int32),
                       pltpu.VMEM((TPW, D), jnp.float32),
                       pltpu.VMEM((TPW, D), jnp.float32),
                       pltpu.SemaphoreType.DMA,
                       pltpu.SemaphoreType.DMA])
    def combine_sc(y_hbm, pos0_hbm, pos1_hbm, out_hbm, i0_v, i1_v, r0_v, r1_v,
                   sem0, sem1):
        base = wid() * TPW
        pltpu.sync_copy(pos0_hbm.at[pl.ds(base, TPW)], i0_v)
        pltpu.sync_copy(pos1_hbm.at[pl.ds(base, TPW)], i1_v)
        c0 = pltpu.async_copy(y_hbm.at[i0_v], r0_v, sem0)
        c1 = pltpu.async_copy(y_hbm.at[i1_v], r1_v, sem1)
        c0.wait()
        c1.wait()

        def row(r, carry):
            for c in range(D // 16):
                sl = pl.ds(c * 16, 16)
                r0_v[r, sl] = r0_v[r, sl] + r1_v[r, sl]
            return carry

        lax.fori_loop(0, TPW, row, 0)
        pltpu.sync_copy(r0_v, out_hbm.at[pl.ds(base, TPW)])

    return scatter_sc, gather0_sc, gather1_sc, combine_sc


def kernel(x, Wg, bg, W1, b1, W2, b2):
    xf = x.reshape(T, D)
    logits = xf @ Wg + bg
    pos0, pos1, w0, w1, gid, nbu, nbu16, lbal = _router(logits)
    pos0 = pos0.reshape(-1)
    pos1 = pos1.reshape(-1)

    scatter_sc, gather0_sc, gather1_sc, combine_sc = _sc_kernels()
    tok = jnp.arange(T, dtype=jnp.int32)
    ptok, pwgt = scatter_sc(pos0, pos1, w0.reshape(-1), w1.reshape(-1), tok)
    xp0 = gather0_sc(ptok, xf)
    xp1 = gather1_sc(ptok, xf)

    gid = gid.reshape(-1)
    nbu = nbu.reshape(-1)
    y0 = _ffn_half(0, gid[:NBH], nbu, xp0, W1, b1, W2, b2,
                   pwgt[:PADH].reshape(PADH, 1))
    y = _ffn_half(NBH, gid[NBH:], nbu, xp1, W1, b1, W2, b2,
                  pwgt[PADH:].reshape(PADH, 1), y0)

    out = combine_sc(y, pos0, pos1).reshape(x.shape)
    return out, lbal[0, 0]


# speedup vs baseline: 3.0798x; 1.2185x over previous
import functools

import jax
import jax.numpy as jnp
from jax import lax
from jax.experimental import pallas as pl
from jax.experimental.pallas import tpu as pltpu
from jax.experimental.pallas import tpu_sc as plsc

D = 768
H = 768
E = 64
K = 2
T = 2048
TOT = T * K
BM = 64
NBC = TOT // BM
SMAX = 128
CS = 128
NC, NS = 2, 16
NW = NC * NS
CH = TOT // NW
SCH = T // (NW // K)
TPW = T // NW


def _router_body(logits_ref,
                 pos0_ref, pos1_ref, w0_ref, w1_ref,
                 g_ref, blk_ref, lo_ref, hi_ref, lbal_ref,
                 oh0_scr, oh1_scr):
    logits = logits_ref[...]

    mx = jnp.max(logits, axis=1, keepdims=True)
    ex = jnp.exp(logits - mx)
    probs = ex / jnp.sum(ex, axis=1, keepdims=True)
    p_mean = jnp.mean(probs, axis=0, keepdims=True)

    lane = lax.broadcasted_iota(jnp.int32, (T, E), 1)
    is0 = logits == mx
    e0 = jnp.min(jnp.where(is0, lane, E), axis=1, keepdims=True)
    oh0 = lane == e0
    masked = jnp.where(oh0, -jnp.inf, logits)
    m1 = jnp.max(masked, axis=1, keepdims=True)
    is1 = masked == m1
    e1 = jnp.min(jnp.where(is1, lane, E), axis=1, keepdims=True)
    oh1 = lane == e1

    w0 = 1.0 / (1.0 + jnp.exp(m1 - mx))
    w0_ref[...] = w0
    w1_ref[...] = 1.0 - w0

    oh0f = oh0.astype(jnp.float32)
    oh1f = oh1.astype(jnp.float32)
    oh0_scr[...] = oh0f
    oh1_scr[...] = oh1f
    cnt0 = jnp.sum(oh0f, axis=0, keepdims=True)
    cnt = cnt0 + jnp.sum(oh1f, axis=0, keepdims=True)

    f = cnt / float(T * K)
    lbal_ref[...] = float(E) * jnp.sum(f * p_mean, keepdims=True)

    tri_e = (lax.broadcasted_iota(jnp.int32, (E, E), 0)
             < lax.broadcasted_iota(jnp.int32, (E, E), 1)).astype(jnp.float32)
    poff = jnp.dot(cnt, tri_e, preferred_element_type=jnp.float32,
                   precision=lax.Precision.HIGHEST)
    pend = poff + cnt

    inv = 1.0 / BM
    fb = jnp.floor(poff * inv)
    lb = jnp.floor((pend - 1.0) * inv)
    nst = jnp.where(cnt > 0.0, lb - fb + 1.0, 0.0)
    sb = jnp.dot(nst, tri_e, preferred_element_type=jnp.float32,
                 precision=lax.Precision.HIGHEST)
    sbe = (sb + nst).astype(jnp.int32)
    stot = jnp.sum(nst, keepdims=True).astype(jnp.int32)

    s_iota = lax.broadcasted_iota(jnp.int32, (SMAX, E), 0)
    cap = jnp.minimum(s_iota, stot - 1)
    e_s = jnp.sum((jnp.broadcast_to(sbe, (SMAX, E)) <= cap).astype(jnp.int32),
                  axis=1, keepdims=True)

    lane_s = lax.broadcasted_iota(jnp.int32, (SMAX, E), 1)
    ohs = (lane_s == e_s).astype(jnp.float32)
    sb_s = jnp.sum(ohs * sb, axis=1, keepdims=True)
    fb_s = jnp.sum(ohs * fb, axis=1, keepdims=True)
    poff_s = jnp.sum(ohs * poff, axis=1, keepdims=True)
    pend_s = jnp.sum(ohs * pend, axis=1, keepdims=True)
    scap = jnp.minimum(lax.broadcasted_iota(jnp.int32, (SMAX, 1), 0),
                       stot - 1).astype(jnp.float32)
    blk_s = fb_s + (scap - sb_s)
    lo_s = jnp.maximum(poff_s - blk_s * BM, 0.0)
    hi_s = jnp.minimum(pend_s - blk_s * BM, float(BM))
    live = (lax.broadcasted_iota(jnp.int32, (SMAX, 1), 0) <= stot - 1)
    lo_s = jnp.where(live, lo_s, 0.0)
    hi_s = jnp.where(live, hi_s, 0.0)

    g_ref[...] = e_s
    blk_ref[...] = blk_s.astype(jnp.int32)
    lo_ref[...] = lo_s.astype(jnp.int32)
    hi_ref[...] = hi_s.astype(jnp.int32)

    tri_m = (lax.broadcasted_iota(jnp.int32, (CS, CS), 1)
             < lax.broadcasted_iota(jnp.int32, (CS, CS), 0)).astype(jnp.float32)
    add0 = poff
    add1 = poff + cnt0

    def chunk(i, carry):
        b0, b1 = carry
        sl = pl.ds(i * CS, CS)
        m0c = oh0_scr[sl, :]
        m1c = oh1_scr[sl, :]
        r0 = jnp.dot(tri_m, m0c, preferred_element_type=jnp.float32,
                     precision=lax.Precision.HIGHEST) + b0
        r1 = jnp.dot(tri_m, m1c, preferred_element_type=jnp.float32,
                     precision=lax.Precision.HIGHEST) + b1
        p0 = jnp.sum((r0 + add0) * m0c, axis=1, keepdims=True)
        p1 = jnp.sum((r1 + add1) * m1c, axis=1, keepdims=True)
        pos0_ref[sl, :] = p0.astype(jnp.int32)
        pos1_ref[sl, :] = p1.astype(jnp.int32)
        return (b0 + jnp.sum(m0c, axis=0, keepdims=True),
                b1 + jnp.sum(m1c, axis=0, keepdims=True))

    zero = jnp.zeros((1, E), jnp.float32)
    lax.fori_loop(0, T // CS, chunk, (zero, zero))


def _router(logits):
    out_shapes = [
        jax.ShapeDtypeStruct((T, 1), jnp.int32),
        jax.ShapeDtypeStruct((T, 1), jnp.int32),
        jax.ShapeDtypeStruct((T, 1), jnp.float32),
        jax.ShapeDtypeStruct((T, 1), jnp.float32),
        jax.ShapeDtypeStruct((SMAX, 1), jnp.int32),
        jax.ShapeDtypeStruct((SMAX, 1), jnp.int32),
        jax.ShapeDtypeStruct((SMAX, 1), jnp.int32),
        jax.ShapeDtypeStruct((SMAX, 1), jnp.int32),
        jax.ShapeDtypeStruct((1, 1), jnp.float32),
    ]
    return pl.pallas_call(
        _router_body,
        out_shape=out_shapes,
        scratch_shapes=[pltpu.VMEM((T, E), jnp.float32),
                        pltpu.VMEM((T, E), jnp.float32)],
    )(logits)


def _ffn_body(g_r, blk_r, lo_r, hi_r,
              x_ref, w1_ref, b1_ref, w2_ref, b2_ref, pw_ref, y_ref, acc):
    s = pl.program_id(0)
    lo = lo_r[s]
    hi = hi_r[s]

    @pl.when(lo < hi)
    def _():
        xb = x_ref[...]
        h = jnp.maximum(
            jnp.dot(xb, w1_ref[0], preferred_element_type=jnp.float32)
            + b1_ref[0], 0.0)
        y = (jnp.dot(h, w2_ref[0], preferred_element_type=jnp.float32)
             + b2_ref[0]) * pw_ref[...]
        row = lax.broadcasted_iota(jnp.int32, (BM, 1), 0)
        mask = (row >= lo) & (row < hi)
        acc[...] = jnp.where(mask, y, acc[...])

    is_last = (s == SMAX - 1) | (blk_r[jnp.minimum(s + 1, SMAX - 1)]
                                 != blk_r[s])

    @pl.when(is_last)
    def _():
        y_ref[...] = acc[...]


def _ffn(g, blk, lo, hi, xs, W1, b1, W2, b2, pwgt):
    grid_spec = pltpu.PrefetchScalarGridSpec(
        num_scalar_prefetch=4,
        grid=(SMAX,),
        in_specs=[
            pl.BlockSpec((BM, D), lambda s, g, blk, lo, hi: (blk[s], 0)),
            pl.BlockSpec((1, D, H), lambda s, g, blk, lo, hi: (g[s], 0, 0)),
            pl.BlockSpec((1, 1, H), lambda s, g, blk, lo, hi: (g[s], 0, 0)),
            pl.BlockSpec((1, H, D), lambda s, g, blk, lo, hi: (g[s], 0, 0)),
            pl.BlockSpec((1, 1, D), lambda s, g, blk, lo, hi: (g[s], 0, 0)),
            pl.BlockSpec((BM, 1), lambda s, g, blk, lo, hi: (blk[s], 0)),
        ],
        out_specs=pl.BlockSpec((BM, D), lambda s, g, blk, lo, hi: (blk[s], 0)),
        scratch_shapes=[pltpu.VMEM((BM, D), jnp.float32)],
    )
    return pl.pallas_call(
        _ffn_body,
        grid_spec=grid_spec,
        out_shape=jax.ShapeDtypeStruct((TOT, D), jnp.float32),
        compiler_params=pltpu.CompilerParams(
            dimension_semantics=("arbitrary",)),
    )(g, blk, lo, hi, xs, W1, b1.reshape(E, 1, H), W2, b2.reshape(E, 1, D),
      pwgt)


@functools.cache
def _sc_kernels():
    mesh = plsc.VectorSubcoreMesh(core_axis_name="c", subcore_axis_name="s")

    def wid():
        return lax.axis_index("s") * NC + lax.axis_index("c")

    @functools.partial(
        pl.kernel, mesh=mesh,
        out_type=[jax.ShapeDtypeStruct((TOT,), jnp.int32),
                  jax.ShapeDtypeStruct((TOT,), jnp.float32)],
        scratch_types=[pltpu.VMEM((SCH,), jnp.int32),
                       pltpu.VMEM((SCH,), jnp.int32),
                       pltpu.VMEM((SCH,), jnp.float32)])
    def scatter_sc(pos0_hbm, pos1_hbm, w0_hbm, w1_hbm, tok_hbm,
                   ptok_hbm, pwgt_hbm, idx_v, tok_v, val_v):
        w = wid()
        c = w // 2
        sl = pl.ds(c * SCH, SCH)
        pltpu.sync_copy(tok_hbm.at[sl], tok_v)

        @pl.when(w % 2 == 0)
        def _():
            pltpu.sync_copy(pos0_hbm.at[sl], idx_v)
            pltpu.sync_copy(w0_hbm.at[sl], val_v)

        @pl.when(w % 2 == 1)
        def _():
            pltpu.sync_copy(pos1_hbm.at[sl], idx_v)
            pltpu.sync_copy(w1_hbm.at[sl], val_v)

        pltpu.sync_copy(tok_v, ptok_hbm.at[idx_v])
        pltpu.sync_copy(val_v, pwgt_hbm.at[idx_v])

    @functools.partial(
        pl.kernel, mesh=mesh,
        out_type=jax.ShapeDtypeStruct((TOT, D), jnp.float32),
        scratch_types=[pltpu.VMEM((CH,), jnp.int32),
                       pltpu.VMEM((CH, D), jnp.float32),
                       pltpu.SemaphoreType.DMA])
    def gather_sc(ptok_hbm, x_hbm, xs_hbm, idx_v, rows_v, sem):
        base = wid() * CH
        pltpu.sync_copy(ptok_hbm.at[pl.ds(base, CH)], idx_v)
        pltpu.async_copy(x_hbm.at[idx_v], rows_v, sem).wait()
        pltpu.sync_copy(rows_v, xs_hbm.at[pl.ds(base, CH)])

    @functools.partial(
        pl.kernel, mesh=mesh,
        out_type=jax.ShapeDtypeStruct((T, D), jnp.float32),
        scratch_types=[pltpu.VMEM((TPW,), jnp.int32),
                       pltpu.VMEM((TPW,), jnp.int32),
                       pltpu.VMEM((TPW, D), jnp.float32),
                       pltpu.VMEM((TPW, D), jnp.float32),
                       pltpu.SemaphoreType.DMA,
                       pltpu.SemaphoreType.DMA])
    def combine_sc(y_hbm, pos0_hbm, pos1_hbm, out_hbm, i0_v, i1_v, r0_v, r1_v,
                   sem0, sem1):
        base = wid() * TPW
        pltpu.sync_copy(pos0_hbm.at[pl.ds(base, TPW)], i0_v)
        pltpu.sync_copy(pos1_hbm.at[pl.ds(base, TPW)], i1_v)
        c0 = pltpu.async_copy(y_hbm.at[i0_v], r0_v, sem0)
        c1 = pltpu.async_copy(y_hbm.at[i1_v], r1_v, sem1)
        c0.wait()
        c1.wait()

        def row(r, carry):
            for c in range(D // 16):
                sl = pl.ds(c * 16, 16)
                r0_v[r, sl] = r0_v[r, sl] + r1_v[r, sl]
            return carry

        lax.fori_loop(0, TPW, row, 0)
        pltpu.sync_copy(r0_v, out_hbm.at[pl.ds(base, TPW)])

    return scatter_sc, gather_sc, combine_sc


def kernel(x, Wg, bg, W1, b1, W2, b2):
    xf = x.reshape(T, D)
    logits = xf @ Wg + bg
    pos0, pos1, w0, w1, g, blk, lo, hi, lbal = _router(logits)
    pos0 = pos0.reshape(-1)
    pos1 = pos1.reshape(-1)

    scatter_sc, gather_sc, combine_sc = _sc_kernels()
    tok = jnp.arange(T, dtype=jnp.int32)
    ptok, pwgt = scatter_sc(pos0, pos1, w0.reshape(-1), w1.reshape(-1), tok)
    xs = gather_sc(ptok, xf)

    y = _ffn(g.reshape(-1), blk.reshape(-1), lo.reshape(-1), hi.reshape(-1),
             xs, W1, b1, W2, b2, pwgt.reshape(TOT, 1))

    out = combine_sc(y, pos0, pos1).reshape(x.shape)
    return out, lbal[0, 0]


# speedup vs baseline: 3.1848x; 1.0341x over previous
import functools

import jax
import jax.numpy as jnp
from jax import lax
from jax.experimental import pallas as pl
from jax.experimental.pallas import tpu as pltpu
from jax.experimental.pallas import tpu_sc as plsc

D = 768
H = 768
E = 64
K = 2
T = 2048
TOT = T * K
BM = 64
NBC = TOT // BM
SMAX = 128
CS = 128
NC, NS = 2, 16
NW = NC * NS
CH = TOT // NW
SCH = T // (NW // K)
TPW = T // NW


def _router_body(logits_ref,
                 pos0_ref, pos1_ref, w0_ref, w1_ref,
                 g_ref, blk_ref, lo_ref, hi_ref, lbal_ref,
                 oh0_scr, oh1_scr):
    logits = logits_ref[...]

    mx = jnp.max(logits, axis=1, keepdims=True)
    ex = jnp.exp(logits - mx)
    probs = ex / jnp.sum(ex, axis=1, keepdims=True)
    p_mean = jnp.mean(probs, axis=0, keepdims=True)

    lane = lax.broadcasted_iota(jnp.int32, (T, E), 1)
    is0 = logits == mx
    e0 = jnp.min(jnp.where(is0, lane, E), axis=1, keepdims=True)
    oh0 = lane == e0
    masked = jnp.where(oh0, -jnp.inf, logits)
    m1 = jnp.max(masked, axis=1, keepdims=True)
    is1 = masked == m1
    e1 = jnp.min(jnp.where(is1, lane, E), axis=1, keepdims=True)
    oh1 = lane == e1

    w0 = 1.0 / (1.0 + jnp.exp(m1 - mx))
    w0_ref[...] = w0
    w1_ref[...] = 1.0 - w0

    oh0f = oh0.astype(jnp.float32)
    oh1f = oh1.astype(jnp.float32)
    oh0_scr[...] = oh0f
    oh1_scr[...] = oh1f
    cnt0 = jnp.sum(oh0f, axis=0, keepdims=True)
    cnt = cnt0 + jnp.sum(oh1f, axis=0, keepdims=True)

    f = cnt / float(T * K)
    lbal_ref[...] = float(E) * jnp.sum(f * p_mean, keepdims=True)

    tri_e = (lax.broadcasted_iota(jnp.int32, (E, E), 0)
             < lax.broadcasted_iota(jnp.int32, (E, E), 1)).astype(jnp.float32)
    poff = jnp.dot(cnt, tri_e, preferred_element_type=jnp.float32,
                   precision=lax.Precision.HIGHEST)
    pend = poff + cnt

    inv = 1.0 / BM
    fb = jnp.floor(poff * inv)
    lb = jnp.floor((pend - 1.0) * inv)
    nst = jnp.where(cnt > 0.0, lb - fb + 1.0, 0.0)
    sb = jnp.dot(nst, tri_e, preferred_element_type=jnp.float32,
                 precision=lax.Precision.HIGHEST)
    sbe = (sb + nst).astype(jnp.int32)
    stot = jnp.sum(nst, keepdims=True).astype(jnp.int32)

    s_iota = lax.broadcasted_iota(jnp.int32, (SMAX, E), 0)
    cap = jnp.minimum(s_iota, stot - 1)
    e_s = jnp.sum((jnp.broadcast_to(sbe, (SMAX, E)) <= cap).astype(jnp.int32),
                  axis=1, keepdims=True)

    lane_s = lax.broadcasted_iota(jnp.int32, (SMAX, E), 1)
    ohs = (lane_s == e_s).astype(jnp.float32)
    sb_s = jnp.sum(ohs * sb, axis=1, keepdims=True)
    fb_s = jnp.sum(ohs * fb, axis=1, keepdims=True)
    poff_s = jnp.sum(ohs * poff, axis=1, keepdims=True)
    pend_s = jnp.sum(ohs * pend, axis=1, keepdims=True)
    scap = jnp.minimum(lax.broadcasted_iota(jnp.int32, (SMAX, 1), 0),
                       stot - 1).astype(jnp.float32)
    blk_s = fb_s + (scap - sb_s)
    lo_s = jnp.maximum(poff_s - blk_s * BM, 0.0)
    hi_s = jnp.minimum(pend_s - blk_s * BM, float(BM))
    live = (lax.broadcasted_iota(jnp.int32, (SMAX, 1), 0) <= stot - 1)
    lo_s = jnp.where(live, lo_s, 0.0)
    hi_s = jnp.where(live, hi_s, 0.0)

    g_ref[...] = e_s
    blk_ref[...] = blk_s.astype(jnp.int32)
    lo_ref[...] = lo_s.astype(jnp.int32)
    hi_ref[...] = hi_s.astype(jnp.int32)

    tri_m = (lax.broadcasted_iota(jnp.int32, (CS, CS), 1)
             < lax.broadcasted_iota(jnp.int32, (CS, CS), 0)).astype(jnp.float32)
    add0 = poff
    add1 = poff + cnt0

    def chunk(i, carry):
        b0, b1 = carry
        sl = pl.ds(i * CS, CS)
        m0c = oh0_scr[sl, :]
        m1c = oh1_scr[sl, :]
        r0 = jnp.dot(tri_m, m0c, preferred_element_type=jnp.float32) + b0
        r1 = jnp.dot(tri_m, m1c, preferred_element_type=jnp.float32) + b1
        p0 = jnp.sum((r0 + add0) * m0c, axis=1, keepdims=True)
        p1 = jnp.sum((r1 + add1) * m1c, axis=1, keepdims=True)
        pos0_ref[sl, :] = p0.astype(jnp.int32)
        pos1_ref[sl, :] = p1.astype(jnp.int32)
        return (b0 + jnp.sum(m0c, axis=0, keepdims=True),
                b1 + jnp.sum(m1c, axis=0, keepdims=True))

    zero = jnp.zeros((1, E), jnp.float32)
    lax.fori_loop(0, T // CS, chunk, (zero, zero))


def _router(logits):
    out_shapes = [
        jax.ShapeDtypeStruct((T, 1), jnp.int32),
        jax.ShapeDtypeStruct((T, 1), jnp.int32),
        jax.ShapeDtypeStruct((T, 1), jnp.float32),
        jax.ShapeDtypeStruct((T, 1), jnp.float32),
        jax.ShapeDtypeStruct((SMAX, 1), jnp.int32),
        jax.ShapeDtypeStruct((SMAX, 1), jnp.int32),
        jax.ShapeDtypeStruct((SMAX, 1), jnp.int32),
        jax.ShapeDtypeStruct((SMAX, 1), jnp.int32),
        jax.ShapeDtypeStruct((1, 1), jnp.float32),
    ]
    return pl.pallas_call(
        _router_body,
        out_shape=out_shapes,
        scratch_shapes=[pltpu.VMEM((T, E), jnp.float32),
                        pltpu.VMEM((T, E), jnp.float32)],
    )(logits)


def _ffn_body(g_r, blk_r, lo_r, hi_r,
              x_ref, w1_ref, b1_ref, w2_ref, b2_ref, pw_ref, y_ref, acc):
    s = pl.program_id(0)
    lo = lo_r[s]
    hi = hi_r[s]

    @pl.when(lo < hi)
    def _():
        xb = x_ref[...]
        h = jnp.maximum(
            jnp.dot(xb, w1_ref[0], preferred_element_type=jnp.float32)
            + b1_ref[0], 0.0)
        y = (jnp.dot(h, w2_ref[0], preferred_element_type=jnp.float32)
             + b2_ref[0]) * pw_ref[...]
        row = lax.broadcasted_iota(jnp.int32, (BM, 1), 0)
        mask = (row >= lo) & (row < hi)
        acc[...] = jnp.where(mask, y, acc[...])

    is_last = (s == SMAX - 1) | (blk_r[jnp.minimum(s + 1, SMAX - 1)]
                                 != blk_r[s])

    @pl.when(is_last)
    def _():
        y_ref[...] = acc[...]


def _ffn(g, blk, lo, hi, xs, W1, b1, W2, b2, pwgt):
    grid_spec = pltpu.PrefetchScalarGridSpec(
        num_scalar_prefetch=4,
        grid=(SMAX,),
        in_specs=[
            pl.BlockSpec((BM, D), lambda s, g, blk, lo, hi: (blk[s], 0)),
            pl.BlockSpec((1, D, H), lambda s, g, blk, lo, hi: (g[s], 0, 0)),
            pl.BlockSpec((1, 1, H), lambda s, g, blk, lo, hi: (g[s], 0, 0)),
            pl.BlockSpec((1, H, D), lambda s, g, blk, lo, hi: (g[s], 0, 0)),
            pl.BlockSpec((1, 1, D), lambda s, g, blk, lo, hi: (g[s], 0, 0)),
            pl.BlockSpec((BM, 1), lambda s, g, blk, lo, hi: (blk[s], 0)),
        ],
        out_specs=pl.BlockSpec((BM, D), lambda s, g, blk, lo, hi: (blk[s], 0)),
        scratch_shapes=[pltpu.VMEM((BM, D), jnp.float32)],
    )
    return pl.pallas_call(
        _ffn_body,
        grid_spec=grid_spec,
        out_shape=jax.ShapeDtypeStruct((TOT, D), jnp.float32),
        compiler_params=pltpu.CompilerParams(
            dimension_semantics=("arbitrary",)),
    )(g, blk, lo, hi, xs, W1, b1.reshape(E, 1, H), W2, b2.reshape(E, 1, D),
      pwgt)


@functools.cache
def _sc_kernels():
    mesh = plsc.VectorSubcoreMesh(core_axis_name="c", subcore_axis_name="s")

    def wid():
        return lax.axis_index("s") * NC + lax.axis_index("c")

    @functools.partial(
        pl.kernel, mesh=mesh,
        out_type=[jax.ShapeDtypeStruct((TOT,), jnp.int32),
                  jax.ShapeDtypeStruct((TOT,), jnp.float32)],
        scratch_types=[pltpu.VMEM((SCH,), jnp.int32),
                       pltpu.VMEM((SCH,), jnp.int32),
                       pltpu.VMEM((SCH,), jnp.float32),
                       pltpu.SemaphoreType.DMA,
                       pltpu.SemaphoreType.DMA,
                       pltpu.SemaphoreType.DMA,
                       pltpu.SemaphoreType.DMA,
                       pltpu.SemaphoreType.DMA])
    def scatter_sc(pos0_hbm, pos1_hbm, w0_hbm, w1_hbm, tok_hbm,
                   ptok_hbm, pwgt_hbm, idx_v, tok_v, val_v,
                   sa, sb, sc, sd, se):
        w = wid()
        c = w // 2
        sl = pl.ds(c * SCH, SCH)
        pltpu.async_copy(tok_hbm.at[sl], tok_v, sa)

        @pl.when(w % 2 == 0)
        def _():
            pltpu.async_copy(pos0_hbm.at[sl], idx_v, sb)
            pltpu.async_copy(w0_hbm.at[sl], val_v, sc)

        @pl.when(w % 2 == 1)
        def _():
            pltpu.async_copy(pos1_hbm.at[sl], idx_v, sb)
            pltpu.async_copy(w1_hbm.at[sl], val_v, sc)

        pltpu.make_async_copy(tok_hbm.at[sl], tok_v, sa).wait()
        pltpu.make_async_copy(pos0_hbm.at[sl], idx_v, sb).wait()
        pltpu.make_async_copy(w0_hbm.at[sl], val_v, sc).wait()
        c0 = pltpu.async_copy(tok_v, ptok_hbm.at[idx_v], sd)
        c1 = pltpu.async_copy(val_v, pwgt_hbm.at[idx_v], se)
        c0.wait()
        c1.wait()

    @functools.partial(
        pl.kernel, mesh=mesh,
        out_type=jax.ShapeDtypeStruct((TOT, D), jnp.float32),
        scratch_types=[pltpu.VMEM((CH,), jnp.int32),
                       pltpu.VMEM((CH, D), jnp.float32),
                       pltpu.SemaphoreType.DMA])
    def gather_sc(ptok_hbm, x_hbm, xs_hbm, idx_v, rows_v, sem):
        base = wid() * CH
        pltpu.sync_copy(ptok_hbm.at[pl.ds(base, CH)], idx_v)
        pltpu.async_copy(x_hbm.at[idx_v], rows_v, sem).wait()
        pltpu.sync_copy(rows_v, xs_hbm.at[pl.ds(base, CH)])

    @functools.partial(
        pl.kernel, mesh=mesh,
        out_type=jax.ShapeDtypeStruct((T, D), jnp.float32),
        scratch_types=[pltpu.VMEM((TPW,), jnp.int32),
                       pltpu.VMEM((TPW,), jnp.int32),
                       pltpu.VMEM((TPW, D), jnp.float32),
                       pltpu.VMEM((TPW, D), jnp.float32),
                       pltpu.SemaphoreType.DMA,
                       pltpu.SemaphoreType.DMA])
    def combine_sc(y_hbm, pos0_hbm, pos1_hbm, out_hbm, i0_v, i1_v, r0_v, r1_v,
                   sem0, sem1):
        base = wid() * TPW
        pltpu.sync_copy(pos0_hbm.at[pl.ds(base, TPW)], i0_v)
        pltpu.sync_copy(pos1_hbm.at[pl.ds(base, TPW)], i1_v)
        c0 = pltpu.async_copy(y_hbm.at[i0_v], r0_v, sem0)
        c1 = pltpu.async_copy(y_hbm.at[i1_v], r1_v, sem1)
        c0.wait()
        c1.wait()

        def row(r, carry):
            for c in range(D // 16):
                sl = pl.ds(c * 16, 16)
                r0_v[r, sl] = r0_v[r, sl] + r1_v[r, sl]
            return carry

        lax.fori_loop(0, TPW, row, 0)
        pltpu.sync_copy(r0_v, out_hbm.at[pl.ds(base, TPW)])

    return scatter_sc, gather_sc, combine_sc


def kernel(x, Wg, bg, W1, b1, W2, b2):
    xf = x.reshape(T, D)
    logits = xf @ Wg + bg
    pos0, pos1, w0, w1, g, blk, lo, hi, lbal = _router(logits)
    pos0 = pos0.reshape(-1)
    pos1 = pos1.reshape(-1)

    scatter_sc, gather_sc, combine_sc = _sc_kernels()
    tok = jnp.arange(T, dtype=jnp.int32)
    ptok, pwgt = scatter_sc(pos0, pos1, w0.reshape(-1), w1.reshape(-1), tok)
    xs = gather_sc(ptok, xf)

    y = _ffn(g.reshape(-1), blk.reshape(-1), lo.reshape(-1), hi.reshape(-1),
             xs, W1, b1, W2, b2, pwgt.reshape(TOT, 1))

    out = combine_sc(y, pos0, pos1).reshape(x.shape)
    return out, lbal[0, 0]


# speedup vs baseline: 3.8191x; 1.1991x over previous
import functools

import jax
import jax.numpy as jnp
from jax import lax
from jax.experimental import pallas as pl
from jax.experimental.pallas import tpu as pltpu
from jax.experimental.pallas import tpu_sc as plsc

D = 768
H = 768
E = 64
K = 2
T = 2048
TOT = T * K
BM = 128
NBC = TOT // BM
SMAX = 96
CS = 256
NC, NS = 2, 16
NW = NC * NS
CH = TOT // NW
SCH = T // (NW // K)
TPW = T // NW


def _router_body(logits_ref,
                 pos0_ref, pos1_ref, w0_ref, w1_ref,
                 g_ref, blk_ref, lo_ref, hi_ref, lbal_ref,
                 oh0_scr, oh1_scr):
    logits = logits_ref[...]

    mx = jnp.max(logits, axis=1, keepdims=True)
    ex = jnp.exp(logits - mx)
    probs = ex / jnp.sum(ex, axis=1, keepdims=True)
    p_mean = jnp.mean(probs, axis=0, keepdims=True)

    lane = lax.broadcasted_iota(jnp.int32, (T, E), 1)
    is0 = logits == mx
    e0 = jnp.min(jnp.where(is0, lane, E), axis=1, keepdims=True)
    oh0 = lane == e0
    masked = jnp.where(oh0, -jnp.inf, logits)
    m1 = jnp.max(masked, axis=1, keepdims=True)
    is1 = masked == m1
    e1 = jnp.min(jnp.where(is1, lane, E), axis=1, keepdims=True)
    oh1 = lane == e1

    w0 = 1.0 / (1.0 + jnp.exp(m1 - mx))
    w0_ref[...] = w0
    w1_ref[...] = 1.0 - w0

    oh0f = oh0.astype(jnp.float32)
    oh1f = oh1.astype(jnp.float32)
    oh0_scr[...] = oh0f
    oh1_scr[...] = oh1f
    cnt0 = jnp.sum(oh0f, axis=0, keepdims=True)
    cnt = cnt0 + jnp.sum(oh1f, axis=0, keepdims=True)

    f = cnt / float(T * K)
    lbal_ref[...] = float(E) * jnp.sum(f * p_mean, keepdims=True)

    tri_e = (lax.broadcasted_iota(jnp.int32, (E, E), 0)
             < lax.broadcasted_iota(jnp.int32, (E, E), 1)).astype(jnp.float32)
    poff = jnp.dot(cnt, tri_e, preferred_element_type=jnp.float32,
                   precision=lax.Precision.HIGHEST)
    pend = poff + cnt

    inv = 1.0 / BM
    fb = jnp.floor(poff * inv)
    lb = jnp.floor((pend - 1.0) * inv)
    nst = jnp.where(cnt > 0.0, lb - fb + 1.0, 0.0)
    sb = jnp.dot(nst, tri_e, preferred_element_type=jnp.float32,
                 precision=lax.Precision.HIGHEST)
    sbe = (sb + nst).astype(jnp.int32)
    stot = jnp.sum(nst, keepdims=True).astype(jnp.int32)

    s_iota = lax.broadcasted_iota(jnp.int32, (SMAX, E), 0)
    cap = jnp.minimum(s_iota, stot - 1)
    e_s = jnp.sum((jnp.broadcast_to(sbe, (SMAX, E)) <= cap).astype(jnp.int32),
                  axis=1, keepdims=True)

    lane_s = lax.broadcasted_iota(jnp.int32, (SMAX, E), 1)
    ohs = (lane_s == e_s).astype(jnp.float32)
    sb_s = jnp.sum(ohs * sb, axis=1, keepdims=True)
    fb_s = jnp.sum(ohs * fb, axis=1, keepdims=True)
    poff_s = jnp.sum(ohs * poff, axis=1, keepdims=True)
    pend_s = jnp.sum(ohs * pend, axis=1, keepdims=True)
    scap = jnp.minimum(lax.broadcasted_iota(jnp.int32, (SMAX, 1), 0),
                       stot - 1).astype(jnp.float32)
    blk_s = fb_s + (scap - sb_s)
    lo_s = jnp.maximum(poff_s - blk_s * BM, 0.0)
    hi_s = jnp.minimum(pend_s - blk_s * BM, float(BM))
    live = (lax.broadcasted_iota(jnp.int32, (SMAX, 1), 0) <= stot - 1)
    lo_s = jnp.where(live, lo_s, 0.0)
    hi_s = jnp.where(live, hi_s, 0.0)

    g_ref[...] = e_s
    blk_ref[...] = blk_s.astype(jnp.int32)
    lo_ref[...] = lo_s.astype(jnp.int32)
    hi_ref[...] = hi_s.astype(jnp.int32)

    tri_m = (lax.broadcasted_iota(jnp.int32, (CS, CS), 1)
             < lax.broadcasted_iota(jnp.int32, (CS, CS), 0)).astype(jnp.float32)
    add0 = poff
    add1 = poff + cnt0

    def chunk(i, carry):
        b0, b1 = carry
        sl = pl.ds(i * CS, CS)
        m0c = oh0_scr[sl, :]
        m1c = oh1_scr[sl, :]
        r0 = jnp.dot(tri_m, m0c, preferred_element_type=jnp.float32) + b0
        r1 = jnp.dot(tri_m, m1c, preferred_element_type=jnp.float32) + b1
        p0 = jnp.sum((r0 + add0) * m0c, axis=1, keepdims=True)
        p1 = jnp.sum((r1 + add1) * m1c, axis=1, keepdims=True)
        pos0_ref[sl, :] = p0.astype(jnp.int32)
        pos1_ref[sl, :] = p1.astype(jnp.int32)
        return (b0 + jnp.sum(m0c, axis=0, keepdims=True),
                b1 + jnp.sum(m1c, axis=0, keepdims=True))

    zero = jnp.zeros((1, E), jnp.float32)
    lax.fori_loop(0, T // CS, chunk, (zero, zero))


def _router(logits):
    out_shapes = [
        jax.ShapeDtypeStruct((T, 1), jnp.int32),
        jax.ShapeDtypeStruct((T, 1), jnp.int32),
        jax.ShapeDtypeStruct((T, 1), jnp.float32),
        jax.ShapeDtypeStruct((T, 1), jnp.float32),
        jax.ShapeDtypeStruct((SMAX, 1), jnp.int32),
        jax.ShapeDtypeStruct((SMAX, 1), jnp.int32),
        jax.ShapeDtypeStruct((SMAX, 1), jnp.int32),
        jax.ShapeDtypeStruct((SMAX, 1), jnp.int32),
        jax.ShapeDtypeStruct((1, 1), jnp.float32),
    ]
    return pl.pallas_call(
        _router_body,
        out_shape=out_shapes,
        scratch_shapes=[pltpu.VMEM((T, E), jnp.float32),
                        pltpu.VMEM((T, E), jnp.float32)],
    )(logits)


def _ffn_body(g_r, blk_r, lo_r, hi_r,
              x_ref, w1_ref, b1_ref, w2_ref, b2_ref, pw_ref, y_ref, acc):
    s = pl.program_id(0)
    lo = lo_r[s]
    hi = hi_r[s]

    @pl.when(lo < hi)
    def _():
        xb = x_ref[...]
        h = jnp.maximum(
            jnp.dot(xb, w1_ref[0], preferred_element_type=jnp.float32)
            + b1_ref[0], 0.0)
        y = (jnp.dot(h, w2_ref[0], preferred_element_type=jnp.float32)
             + b2_ref[0]) * pw_ref[...]
        row = lax.broadcasted_iota(jnp.int32, (BM, 1), 0)
        mask = (row >= lo) & (row < hi)
        acc[...] = jnp.where(mask, y, acc[...])

    is_last = (s == SMAX - 1) | (blk_r[jnp.minimum(s + 1, SMAX - 1)]
                                 != blk_r[s])

    @pl.when(is_last)
    def _():
        y_ref[...] = acc[...]


def _ffn(g, blk, lo, hi, xs, W1, b1, W2, b2, pwgt):
    grid_spec = pltpu.PrefetchScalarGridSpec(
        num_scalar_prefetch=4,
        grid=(SMAX,),
        in_specs=[
            pl.BlockSpec((BM, D), lambda s, g, blk, lo, hi: (blk[s], 0)),
            pl.BlockSpec((1, D, H), lambda s, g, blk, lo, hi: (g[s], 0, 0)),
            pl.BlockSpec((1, 1, H), lambda s, g, blk, lo, hi: (g[s], 0, 0)),
            pl.BlockSpec((1, H, D), lambda s, g, blk, lo, hi: (g[s], 0, 0)),
            pl.BlockSpec((1, 1, D), lambda s, g, blk, lo, hi: (g[s], 0, 0)),
            pl.BlockSpec((BM, 1), lambda s, g, blk, lo, hi: (blk[s], 0)),
        ],
        out_specs=pl.BlockSpec((BM, D), lambda s, g, blk, lo, hi: (blk[s], 0)),
        scratch_shapes=[pltpu.VMEM((BM, D), jnp.float32)],
    )
    return pl.pallas_call(
        _ffn_body,
        grid_spec=grid_spec,
        out_shape=jax.ShapeDtypeStruct((TOT, D), jnp.float32),
        compiler_params=pltpu.CompilerParams(
            dimension_semantics=("arbitrary",)),
    )(g, blk, lo, hi, xs, W1, b1.reshape(E, 1, H), W2, b2.reshape(E, 1, D),
      pwgt)


@functools.cache
def _sc_kernels():
    mesh = plsc.VectorSubcoreMesh(core_axis_name="c", subcore_axis_name="s")

    def wid():
        return lax.axis_index("s") * NC + lax.axis_index("c")

    @functools.partial(
        pl.kernel, mesh=mesh,
        out_type=[jax.ShapeDtypeStruct((TOT,), jnp.int32),
                  jax.ShapeDtypeStruct((TOT,), jnp.float32)],
        scratch_types=[pltpu.VMEM((SCH,), jnp.int32),
                       pltpu.VMEM((SCH,), jnp.int32),
                       pltpu.VMEM((SCH,), jnp.float32),
                       pltpu.SemaphoreType.DMA,
                       pltpu.SemaphoreType.DMA,
                       pltpu.SemaphoreType.DMA,
                       pltpu.SemaphoreType.DMA,
                       pltpu.SemaphoreType.DMA])
    def scatter_sc(pos0_hbm, pos1_hbm, w0_hbm, w1_hbm, tok_hbm,
                   ptok_hbm, pwgt_hbm, idx_v, tok_v, val_v,
                   sa, sb, sc, sd, se):
        w = wid()
        c = w // 2
        sl = pl.ds(c * SCH, SCH)
        pltpu.async_copy(tok_hbm.at[sl], tok_v, sa)

        @pl.when(w % 2 == 0)
        def _():
            pltpu.async_copy(pos0_hbm.at[sl], idx_v, sb)
            pltpu.async_copy(w0_hbm.at[sl], val_v, sc)

        @pl.when(w % 2 == 1)
        def _():
            pltpu.async_copy(pos1_hbm.at[sl], idx_v, sb)
            pltpu.async_copy(w1_hbm.at[sl], val_v, sc)

        pltpu.make_async_copy(tok_hbm.at[sl], tok_v, sa).wait()
        pltpu.make_async_copy(pos0_hbm.at[sl], idx_v, sb).wait()
        pltpu.make_async_copy(w0_hbm.at[sl], val_v, sc).wait()
        c0 = pltpu.async_copy(tok_v, ptok_hbm.at[idx_v], sd)
        c1 = pltpu.async_copy(val_v, pwgt_hbm.at[idx_v], se)
        c0.wait()
        c1.wait()

    @functools.partial(
        pl.kernel, mesh=mesh,
        out_type=jax.ShapeDtypeStruct((TOT, D), jnp.float32),
        scratch_types=[pltpu.VMEM((CH,), jnp.int32),
                       pltpu.VMEM((CH, D), jnp.float32),
                       pltpu.SemaphoreType.DMA])
    def gather_sc(ptok_hbm, x_hbm, xs_hbm, idx_v, rows_v, sem):
        base = wid() * CH
        pltpu.sync_copy(ptok_hbm.at[pl.ds(base, CH)], idx_v)
        pltpu.async_copy(x_hbm.at[idx_v], rows_v, sem).wait()
        pltpu.sync_copy(rows_v, xs_hbm.at[pl.ds(base, CH)])

    @functools.partial(
        pl.kernel, mesh=mesh,
        out_type=jax.ShapeDtypeStruct((T, D), jnp.float32),
        scratch_types=[pltpu.VMEM((TPW,), jnp.int32),
                       pltpu.VMEM((TPW,), jnp.int32),
                       pltpu.VMEM((TPW, D), jnp.float32),
                       pltpu.VMEM((TPW, D), jnp.float32),
                       pltpu.SemaphoreType.DMA,
                       pltpu.SemaphoreType.DMA])
    def combine_sc(y_hbm, pos0_hbm, pos1_hbm, out_hbm, i0_v, i1_v, r0_v, r1_v,
                   sem0, sem1):
        base = wid() * TPW
        pltpu.sync_copy(pos0_hbm.at[pl.ds(base, TPW)], i0_v)
        pltpu.sync_copy(pos1_hbm.at[pl.ds(base, TPW)], i1_v)
        c0 = pltpu.async_copy(y_hbm.at[i0_v], r0_v, sem0)
        c1 = pltpu.async_copy(y_hbm.at[i1_v], r1_v, sem1)
        c0.wait()
        c1.wait()

        def row(r, carry):
            for c in range(D // 16):
                sl = pl.ds(c * 16, 16)
                r0_v[r, sl] = r0_v[r, sl] + r1_v[r, sl]
            return carry

        lax.fori_loop(0, TPW, row, 0)
        pltpu.sync_copy(r0_v, out_hbm.at[pl.ds(base, TPW)])

    return scatter_sc, gather_sc, combine_sc


def kernel(x, Wg, bg, W1, b1, W2, b2):
    xf = x.reshape(T, D)
    logits = xf @ Wg + bg
    pos0, pos1, w0, w1, g, blk, lo, hi, lbal = _router(logits)
    pos0 = pos0.reshape(-1)
    pos1 = pos1.reshape(-1)

    scatter_sc, gather_sc, combine_sc = _sc_kernels()
    tok = jnp.arange(T, dtype=jnp.int32)
    ptok, pwgt = scatter_sc(pos0, pos1, w0.reshape(-1), w1.reshape(-1), tok)
    xs = gather_sc(ptok, xf)

    y = _ffn(g.reshape(-1), blk.reshape(-1), lo.reshape(-1), hi.reshape(-1),
             xs, W1, b1, W2, b2, pwgt.reshape(TOT, 1))

    out = combine_sc(y, pos0, pos1).reshape(x.shape)
    return out, lbal[0, 0]
